# Initial kernel scaffold; baseline (speedup 1.0000x reference)
import jax
import jax.numpy as jnp
from jax.experimental import pallas as pl


def kernel(x, pos, edge_attr, u, params, edge_index, motif_types):
    raise NotImplementedError("write your pallas kernel here")


# speedup vs baseline: 4.1073x; 4.1073x over previous
import functools
import math

import jax
import jax.numpy as jnp
from jax import lax
from jax.experimental import pallas as pl
from jax.experimental.pallas import tpu as pltpu
from jax.experimental.pallas import tpu_sc as plsc

N_NODES = 10000
N_EDGES = 160000
NODE_DIM = 128
EDGE_DIM = 16
HIDDEN_DIM = 128
NUM_RBF = 64
CUTOFF = 10.0

AUG = 144
AUXW = 16
NC, NS = 2, 16
NW = NC * NS
CH = 128
NCHUNK = N_EDGES // CH
CPW = (NCHUNK + NW - 1) // NW
RPS = N_NODES // NS

EB = 640
NB = 1000

def _sc_mesh():
    return plsc.VectorSubcoreMesh(core_axis_name="c", subcore_axis_name="s",
                                  num_cores=NC, num_subcores=NS)


@functools.cache
def _make_sc_gather():
    return functools.partial(
        pl.kernel,
        out_type=(jax.ShapeDtypeStruct((N_EDGES, AUG), jnp.float32),
                  jax.ShapeDtypeStruct((N_EDGES, AUG), jnp.float32)),
        mesh=_sc_mesh(),
        scratch_types=[pltpu.VMEM((CH,), jnp.int32),
                       pltpu.VMEM((CH,), jnp.int32),
                       pltpu.VMEM((CH, AUG), jnp.float32),
                       pltpu.VMEM((CH, AUG), jnp.float32),
                       pltpu.SemaphoreType.DMA,
                       pltpu.SemaphoreType.DMA],
        compiler_params=pltpu.CompilerParams(use_tc_tiling_on_sc=False),
    )(_sc_gather_body)


def _sc_gather_body(tab, rowi, coli, gi, gj, idxa, idxb, bufa, bufb, sema, semb):
    wid = lax.axis_index("s") * NC + lax.axis_index("c")

    def body(k, carry):
        c = k * NW + wid

        @pl.when(c < NCHUNK)
        def _():
            off = c * CH
            pltpu.sync_copy(rowi.at[pl.ds(off, CH)], idxa)
            pltpu.sync_copy(coli.at[pl.ds(off, CH)], idxb)
            ca = pltpu.async_copy(tab.at[idxa], bufa, sema)
            cb = pltpu.async_copy(tab.at[idxb], bufb, semb)
            ca.wait()
            pltpu.sync_copy(bufa, gi.at[pl.ds(off, CH)])
            cb.wait()
            pltpu.sync_copy(bufb, gj.at[pl.ds(off, CH)])

        return carry

    lax.fori_loop(0, CPW, body, 0)


@functools.cache
def _make_sc_scatter():
    return functools.partial(
        pl.kernel,
        out_type=(jax.ShapeDtypeStruct((NC, N_NODES, HIDDEN_DIM), jnp.float32),
                  jax.ShapeDtypeStruct((NC, N_NODES, AUXW), jnp.float32)),
        mesh=_sc_mesh(),
        scratch_types=[pltpu.VMEM((CH,), jnp.int32),
                       pltpu.VMEM((CH, HIDDEN_DIM), jnp.float32),
                       pltpu.VMEM((CH, AUXW), jnp.float32),
                       pltpu.VMEM_SHARED((N_NODES, HIDDEN_DIM), jnp.float32),
                       pltpu.VMEM_SHARED((N_NODES, AUXW), jnp.float32)],
        compiler_params=pltpu.CompilerParams(use_tc_tiling_on_sc=False),
    )(_sc_scatter_body)


def _sc_scatter_body(rowi, me, aux, zm, za, pm, pa, idx, mbuf, abuf, macc, aacc):
    cid = lax.axis_index("c")
    sid = lax.axis_index("s")
    wid = sid * NC + cid
    r0 = sid * RPS
    pltpu.sync_copy(zm.at[pl.ds(r0, RPS)], macc.at[pl.ds(r0, RPS)])
    pltpu.sync_copy(za.at[pl.ds(r0, RPS)], aacc.at[pl.ds(r0, RPS)])
    plsc.subcore_barrier()

    def body(k, carry):
        c = k * NW + wid

        @pl.when(c < NCHUNK)
        def _():
            off = c * CH
            pltpu.sync_copy(rowi.at[pl.ds(off, CH)], idx)
            pltpu.sync_copy(me.at[pl.ds(off, CH)], mbuf)
            pltpu.sync_copy(aux.at[pl.ds(off, CH)], abuf)
            pltpu.sync_copy(mbuf, macc.at[idx], add=True)
            pltpu.sync_copy(abuf, aacc.at[idx], add=True)

        return carry

    lax.fori_loop(0, CPW, body, 0)
    plsc.subcore_barrier()
    pltpu.sync_copy(macc.at[pl.ds(r0, RPS)], pm.at[cid, pl.ds(r0, RPS)])
    pltpu.sync_copy(aacc.at[pl.ds(r0, RPS)], pa.at[cid, pl.ds(r0, RPS)])


def _silu(v):
    return v * jax.nn.sigmoid(v)


def _ln(v, g, b):
    mu = jnp.mean(v, axis=-1, keepdims=True)
    d = v - mu
    var = jnp.mean(d * d, axis=-1, keepdims=True)
    return d * jax.lax.rsqrt(var + 1e-5) * g + b


def _dot(a, b):
    return jax.lax.dot_general(a, b, (((1,), (0,)), ((), ())),
                               preferred_element_type=jnp.float32)


def _edge_body(gi, gj, ea, uu,
               w1x, w1y, w1r, w1a, w1u, b1, lg1, lb1, w2, b2, w3, b3,
               mw1x, mw1y, mw1e, mw1r, mb1, mlg, mlb, mw2, mb2, aw, ab,
               cw1x, cw1y, cw1r, cb1, cw2, cb2,
               c5, means, betas,
               ean_o, me_o, aux_o):
    f32 = jnp.float32
    xi = gi[:, :NODE_DIM]
    xj = gj[:, :NODE_DIM]
    rel0 = gi[:, NODE_DIM:NODE_DIM + 1] - gj[:, NODE_DIM:NODE_DIM + 1]
    rel1 = gi[:, NODE_DIM + 1:NODE_DIM + 2] - gj[:, NODE_DIM + 1:NODE_DIM + 2]
    rel2 = gi[:, NODE_DIM + 2:NODE_DIM + 3] - gj[:, NODE_DIM + 2:NODE_DIM + 3]
    dist = jnp.sqrt(rel0 * rel0 + rel1 * rel1 + rel2 * rel2 + 1e-12)
    expd = jnp.exp((-5.0 / CUTOFF) * dist)
    cut = 0.5 * (jnp.cos(dist * (math.pi / CUTOFF)) + 1.0)
    cut = cut * (dist < CUTOFF).astype(f32)
    rbf = cut * jnp.exp(-betas[...] * (expd - means[...]) ** 2)

    h = (_dot(xi, w1x[...]) + _dot(xj, w1y[...]) + _dot(rbf, w1r[...])
         + _dot(ea[...], w1a[...]) + _dot(uu[...], w1u[...]) + b1[...])
    h = _ln(_silu(h), lg1[...], lb1[...])
    h = _silu(_dot(h, w2[...]) + b2[...])
    ean = _dot(h, w3[...]) + b3[...]
    ean_o[...] = ean

    m = (_dot(xi, mw1x[...]) + _dot(xj, mw1y[...]) + _dot(ean, mw1e[...])
         + _dot(rbf, mw1r[...]) + mb1[...])
    m = _ln(_silu(m), mlg[...], mlb[...])
    m = _dot(m, mw2[...]) + mb2[...]

    logits = jnp.sum(m * aw[...], axis=-1, keepdims=True) + ab[...]
    ti = gi[:, NODE_DIM + 3:NODE_DIM + 4]
    tj = gj[:, NODE_DIM + 3:NODE_DIM + 4]
    io5 = lax.broadcasted_iota(jnp.int32, (1, 5), 1)
    ohi = (ti.astype(jnp.int32) == io5).astype(f32)
    ohj = (tj.astype(jnp.int32) == io5).astype(f32)
    bias = jnp.sum(_dot(ohi, c5[...]) * ohj, axis=-1, keepdims=True)
    e = jnp.exp(logits + bias)
    me_o[...] = m * e

    chh = _silu(_dot(xi, cw1x[...]) + _dot(xj, cw1y[...])
                + _dot(rbf, cw1r[...]) + cb1[...])
    w = jnp.tanh(jnp.sum(chh * cw2[...], axis=-1, keepdims=True) + cb2[...])
    one = jnp.ones_like(e)
    aux_o[...] = jnp.concatenate(
        [e, rel0 * w, rel1 * w, rel2 * w, one,
         jnp.zeros((e.shape[0], AUXW - 5), f32)], axis=1)


def _node_body(x, pos, pm, pa,
               nw1x, nw1a, nb1, nlg, nlb, nw2, nb2, ng, nbb,
               xo, po):
    msum = pm[0] + pm[1]
    aux = pa[0] + pa[1]
    denom = aux[:, 0:1]
    delta = aux[:, 1:4]
    cnt = aux[:, 4:5]
    safe = jnp.where(denom > 0.0, denom, 1.0)
    agg = jnp.where(denom > 0.0, msum / safe, 0.0)
    xv = x[...]
    nh = _silu(_dot(xv, nw1x[...]) + _dot(agg, nw1a[...]) + nb1[...])
    nh = _ln(nh, nlg[...], nlb[...])
    nh = _dot(nh, nw2[...]) + nb2[...]
    xo[...] = _ln(xv + nh, ng[...], nbb[...])
    po[...] = pos[...] + delta / (cnt + 1e-8)


def _row(v):
    return v[None, :]


def kernel(x, pos, edge_attr, u, params, edge_index, motif_types):
    p = params
    f32 = jnp.float32
    row = edge_index[0]
    col = edge_index[1]

    tab = jnp.concatenate(
        [x, pos, motif_types.astype(f32)[:, None],
         jnp.zeros((N_NODES, AUG - NODE_DIM - 4), f32)], axis=1)

    gi, gj = _make_sc_gather()(tab, row, col)

    w1 = p['eu_w1']
    mw1 = p['msg_w1']
    cw1 = p['coord_w1']
    c5 = (p['cross_bias'] + p['motif_imp'][:, None] + p['motif_imp'][None, :])

    eargs = (
        w1[0:128], w1[128:256], w1[256:320], w1[320:336], w1[336:464],
        _row(p['eu_b1']), _row(p['eu_ln_g']), _row(p['eu_ln_b']),
        p['eu_w2'], _row(p['eu_b2']), p['eu_w3'], _row(p['eu_b3']),
        mw1[0:128], mw1[128:256], mw1[256:272], mw1[272:336],
        _row(p['msg_b1']), _row(p['msg_ln_g']), _row(p['msg_ln_b']),
        p['msg_w2'], _row(p['msg_b2']),
        p['attn_w'].T, _row(p['attn_b']),
        cw1[0:128], cw1[128:256], cw1[256:320], _row(p['coord_b1']),
        p['coord_w2'].T, _row(p['coord_b2']),
        c5, _row(p['eu_means']), _row(p['eu_betas']),
    )

    nblk = N_EDGES // EB
    dspec = lambda d: pl.BlockSpec((EB, d), lambda i: (i, 0))
    wspec = lambda a: pl.BlockSpec(a.shape, lambda i: (0,) * a.ndim)

    ean, me, aux = pl.pallas_call(
        _edge_body,
        grid=(nblk,),
        in_specs=[dspec(AUG), dspec(AUG), dspec(EDGE_DIM), dspec(HIDDEN_DIM)]
                 + [wspec(a) for a in eargs],
        out_specs=[dspec(EDGE_DIM), dspec(HIDDEN_DIM), dspec(AUXW)],
        out_shape=[jax.ShapeDtypeStruct((N_EDGES, EDGE_DIM), f32),
                   jax.ShapeDtypeStruct((N_EDGES, HIDDEN_DIM), f32),
                   jax.ShapeDtypeStruct((N_EDGES, AUXW), f32)],
    )(gi, gj, edge_attr, u, *eargs)

    zm = jnp.zeros((N_NODES, HIDDEN_DIM), f32)
    za = jnp.zeros((N_NODES, AUXW), f32)
    pm, pa = _make_sc_scatter()(row, me, aux, zm, za)

    nargs = (
        p['node_w1'][0:128], p['node_w1'][128:256], _row(p['node_b1']),
        _row(p['node_ln_g']), _row(p['node_ln_b']),
        p['node_w2'], _row(p['node_b2']),
        _row(p['norm_g']), _row(p['norm_b']),
    )
    ndspec = lambda d: pl.BlockSpec((NB, d), lambda i: (i, 0))
    pspec = lambda d: pl.BlockSpec((NC, NB, d), lambda i: (0, i, 0))

    x_out, pos_out = pl.pallas_call(
        _node_body,
        grid=(N_NODES // NB,),
        in_specs=[ndspec(NODE_DIM), ndspec(3), pspec(HIDDEN_DIM), pspec(AUXW)]
                 + [wspec(a) for a in nargs],
        out_specs=[ndspec(NODE_DIM), ndspec(3)],
        out_shape=[jax.ShapeDtypeStruct((N_NODES, NODE_DIM), f32),
                   jax.ShapeDtypeStruct((N_NODES, 3), f32)],
    )(x, pos, pm, pa, *nargs)

    return x_out, pos_out, ean


# speedup vs baseline: 4.7646x; 1.1600x over previous
import functools
import math

import jax
import jax.numpy as jnp
from jax import lax
from jax.experimental import pallas as pl
from jax.experimental.pallas import tpu as pltpu
from jax.experimental.pallas import tpu_sc as plsc

N_NODES = 10000
N_EDGES = 160000
NODE_DIM = 128
EDGE_DIM = 16
HIDDEN_DIM = 128
NUM_RBF = 64
CUTOFF = 10.0

AUXW = 16
NC, NS = 2, 16
NW = NC * NS
CH = 128
NCHUNK = N_EDGES // CH
CPW = (NCHUNK + NW - 1) // NW
NPAD = 10240
RPS = NPAD // NS

EB = 640
NB = 1000


def _sc_mesh():
    return plsc.VectorSubcoreMesh(core_axis_name="c", subcore_axis_name="s",
                                  num_cores=NC, num_subcores=NS)


def _gather_body(tab, rowi, coli, gi, gj, idxa, idxb, bufa, bufb, sema, semb):
    wid = lax.axis_index("s") * NC + lax.axis_index("c")

    def body(k, carry):
        c = k * NW + wid

        @pl.when(c < NCHUNK)
        def _():
            off = c * CH
            pltpu.sync_copy(rowi.at[pl.ds(off, CH)], idxa)
            pltpu.sync_copy(coli.at[pl.ds(off, CH)], idxb)
            ca = pltpu.async_copy(tab.at[idxa], bufa, sema)
            cb = pltpu.async_copy(tab.at[idxb], bufb, semb)
            ca.wait()
            pltpu.sync_copy(bufa, gi.at[pl.ds(off, CH)])
            cb.wait()
            pltpu.sync_copy(bufb, gj.at[pl.ds(off, CH)])

        return carry

    lax.fori_loop(0, CPW, body, 0)


@functools.cache
def _make_gather(width, tc_tiling):
    return functools.partial(
        pl.kernel,
        out_type=(jax.ShapeDtypeStruct((N_EDGES, width), jnp.float32),
                  jax.ShapeDtypeStruct((N_EDGES, width), jnp.float32)),
        mesh=_sc_mesh(),
        scratch_types=[pltpu.VMEM((CH,), jnp.int32),
                       pltpu.VMEM((CH,), jnp.int32),
                       pltpu.VMEM((CH, width), jnp.float32),
                       pltpu.VMEM((CH, width), jnp.float32),
                       pltpu.SemaphoreType.DMA,
                       pltpu.SemaphoreType.DMA],
        compiler_params=pltpu.CompilerParams(use_tc_tiling_on_sc=tc_tiling),
    )(_gather_body)


def _scatter_body(rowi, val, zz, po, idx, vbuf, acc):
    cid = lax.axis_index("c")
    sid = lax.axis_index("s")
    wid = sid * NC + cid
    r0 = sid * RPS
    pltpu.sync_copy(zz.at[pl.ds(r0, RPS)], acc.at[pl.ds(r0, RPS)])
    plsc.subcore_barrier()

    def body(k, carry):
        c = k * NW + wid

        @pl.when(c < NCHUNK)
        def _():
            off = c * CH
            pltpu.sync_copy(rowi.at[pl.ds(off, CH)], idx)
            pltpu.sync_copy(val.at[pl.ds(off, CH)], vbuf)
            pltpu.sync_copy(vbuf, acc.at[idx], add=True)

        return carry

    lax.fori_loop(0, CPW, body, 0)
    plsc.subcore_barrier()
    pltpu.sync_copy(acc.at[pl.ds(r0, RPS)], po.at[cid, pl.ds(r0, RPS)])


@functools.cache
def _make_scatter(width, tc_tiling):
    return functools.partial(
        pl.kernel,
        out_type=jax.ShapeDtypeStruct((NC, NPAD, width), jnp.float32),
        mesh=_sc_mesh(),
        scratch_types=[pltpu.VMEM((CH,), jnp.int32),
                       pltpu.VMEM((CH, width), jnp.float32),
                       pltpu.VMEM_SHARED((NPAD, width), jnp.float32)],
        compiler_params=pltpu.CompilerParams(use_tc_tiling_on_sc=tc_tiling),
    )(_scatter_body)


def _silu(v):
    return v * jax.nn.sigmoid(v)


def _ln(v, g, b):
    mu = jnp.mean(v, axis=-1, keepdims=True)
    d = v - mu
    var = jnp.mean(d * d, axis=-1, keepdims=True)
    return d * jax.lax.rsqrt(var + 1e-5) * g + b


def _dot(a, b):
    return jax.lax.dot_general(a, b, (((1,), (0,)), ((), ())),
                               preferred_element_type=jnp.float32)


def _edge_body(xi_r, xj_r, ai_r, aj_r, ea, uu,
               w1x, w1y, w1r, w1a, w1u, b1, lg1, lb1, w2, b2, w3, b3,
               mw1x, mw1y, mw1e, mw1r, mb1, mlg, mlb, mw2, mb2, aw, ab,
               cw1x, cw1y, cw1r, cb1, cw2, cb2,
               c5, means, betas,
               ean_o, me_o, aux_o):
    f32 = jnp.float32
    xi = xi_r[...]
    xj = xj_r[...]
    rel = ai_r[:, 0:3] - aj_r[:, 0:3]
    d2 = jnp.sum(rel * rel, axis=-1, keepdims=True) + 1e-12
    dist = jnp.sqrt(d2)
    expd = jnp.exp((-5.0 / CUTOFF) * dist)
    cut = 0.5 * (jnp.cos(dist * (math.pi / CUTOFF)) + 1.0)
    cut = cut * (dist < CUTOFF).astype(f32)
    rbf = cut * jnp.exp(-betas[...] * (expd - means[...]) ** 2)

    h = (_dot(xi, w1x[...]) + _dot(xj, w1y[...]) + _dot(rbf, w1r[...])
         + _dot(ea[...], w1a[...]) + _dot(uu[...], w1u[...]) + b1[...])
    h = _ln(_silu(h), lg1[...], lb1[...])
    h = _silu(_dot(h, w2[...]) + b2[...])
    ean = _dot(h, w3[...]) + b3[...]
    ean_o[...] = ean

    m = (_dot(xi, mw1x[...]) + _dot(xj, mw1y[...]) + _dot(ean, mw1e[...])
         + _dot(rbf, mw1r[...]) + mb1[...])
    m = _ln(_silu(m), mlg[...], mlb[...])
    m = _dot(m, mw2[...]) + mb2[...]

    logits = jnp.sum(m * aw[...], axis=-1, keepdims=True) + ab[...]
    ti = ai_r[:, 3:4]
    tj = aj_r[:, 3:4]
    io5 = lax.broadcasted_iota(jnp.int32, (1, 5), 1)
    ohi = (ti.astype(jnp.int32) == io5).astype(f32)
    ohj = (tj.astype(jnp.int32) == io5).astype(f32)
    bias = jnp.sum(_dot(ohi, c5[...]) * ohj, axis=-1, keepdims=True)
    e = jnp.exp(logits + bias)
    me_o[...] = m * e

    chh = _silu(_dot(xi, cw1x[...]) + _dot(xj, cw1y[...])
                + _dot(rbf, cw1r[...]) + cb1[...])
    w = jnp.tanh(jnp.sum(chh * cw2[...], axis=-1, keepdims=True) + cb2[...])
    one = jnp.ones_like(e)
    aux_o[...] = jnp.concatenate(
        [e, rel * w, one, jnp.zeros((e.shape[0], AUXW - 5), f32)], axis=1)


def _node_body(x, pos, pm, pa,
               nw1x, nw1a, nb1, nlg, nlb, nw2, nb2, ng, nbb,
               xo, po):
    msum = pm[0] + pm[1]
    aux = pa[0] + pa[1]
    denom = aux[:, 0:1]
    delta = aux[:, 1:4]
    cnt = aux[:, 4:5]
    safe = jnp.where(denom > 0.0, denom, 1.0)
    agg = jnp.where(denom > 0.0, msum / safe, 0.0)
    xv = x[...]
    nh = _silu(_dot(xv, nw1x[...]) + _dot(agg, nw1a[...]) + nb1[...])
    nh = _ln(nh, nlg[...], nlb[...])
    nh = _dot(nh, nw2[...]) + nb2[...]
    xo[...] = _ln(xv + nh, ng[...], nbb[...])
    po[...] = pos[...] + delta / (cnt + 1e-8)


def _row(v):
    return v[None, :]


def kernel(x, pos, edge_attr, u, params, edge_index, motif_types):
    p = params
    f32 = jnp.float32
    row = edge_index[0]
    col = edge_index[1]

    atab = jnp.concatenate(
        [pos, motif_types.astype(f32)[:, None],
         jnp.zeros((N_NODES, AUXW - 4), f32)], axis=1)

    xi, xj = _make_gather(NODE_DIM, True)(x, row, col)
    ai, aj = _make_gather(AUXW, False)(atab, row, col)

    w1 = p['eu_w1']
    mw1 = p['msg_w1']
    cw1 = p['coord_w1']
    c5 = (p['cross_bias'] + p['motif_imp'][:, None] + p['motif_imp'][None, :])

    eargs = (
        w1[0:128], w1[128:256], w1[256:320], w1[320:336], w1[336:464],
        _row(p['eu_b1']), _row(p['eu_ln_g']), _row(p['eu_ln_b']),
        p['eu_w2'], _row(p['eu_b2']), p['eu_w3'], _row(p['eu_b3']),
        mw1[0:128], mw1[128:256], mw1[256:272], mw1[272:336],
        _row(p['msg_b1']), _row(p['msg_ln_g']), _row(p['msg_ln_b']),
        p['msg_w2'], _row(p['msg_b2']),
        p['attn_w'].T, _row(p['attn_b']),
        cw1[0:128], cw1[128:256], cw1[256:320], _row(p['coord_b1']),
        p['coord_w2'].T, _row(p['coord_b2']),
        c5, _row(p['eu_means']), _row(p['eu_betas']),
    )

    nblk = N_EDGES // EB
    dspec = lambda d: pl.BlockSpec((EB, d), lambda i: (i, 0))
    wspec = lambda a: pl.BlockSpec(a.shape, lambda i: (0,) * a.ndim)

    ean, me, aux = pl.pallas_call(
        _edge_body,
        grid=(nblk,),
        in_specs=[dspec(NODE_DIM), dspec(NODE_DIM), dspec(AUXW), dspec(AUXW),
                  dspec(EDGE_DIM), dspec(HIDDEN_DIM)]
                 + [wspec(a) for a in eargs],
        out_specs=[dspec(EDGE_DIM), dspec(HIDDEN_DIM), dspec(AUXW)],
        out_shape=[jax.ShapeDtypeStruct((N_EDGES, EDGE_DIM), f32),
                   jax.ShapeDtypeStruct((N_EDGES, HIDDEN_DIM), f32),
                   jax.ShapeDtypeStruct((N_EDGES, AUXW), f32)],
    )(xi, xj, ai, aj, edge_attr, u, *eargs)

    zm = jnp.zeros((NPAD, HIDDEN_DIM), f32)
    za = jnp.zeros((NPAD, AUXW), f32)
    pm = _make_scatter(HIDDEN_DIM, True)(row, me, zm)
    pa = _make_scatter(AUXW, False)(row, aux, za)

    nargs = (
        p['node_w1'][0:128], p['node_w1'][128:256], _row(p['node_b1']),
        _row(p['node_ln_g']), _row(p['node_ln_b']),
        p['node_w2'], _row(p['node_b2']),
        _row(p['norm_g']), _row(p['norm_b']),
    )
    ndspec = lambda d: pl.BlockSpec((NB, d), lambda i: (i, 0))
    pspec = lambda d: pl.BlockSpec((NC, NB, d), lambda i: (0, i, 0))

    x_out, pos_out = pl.pallas_call(
        _node_body,
        grid=(N_NODES // NB,),
        in_specs=[ndspec(NODE_DIM), ndspec(3), pspec(HIDDEN_DIM), pspec(AUXW)]
                 + [wspec(a) for a in nargs],
        out_specs=[ndspec(NODE_DIM), ndspec(3)],
        out_shape=[jax.ShapeDtypeStruct((N_NODES, NODE_DIM), f32),
                   jax.ShapeDtypeStruct((N_NODES, 3), f32)],
    )(x, pos, pm, pa, *nargs)

    return x_out, pos_out, ean


# speedup vs baseline: 5.1881x; 1.0889x over previous
import functools
import math

import jax
import jax.numpy as jnp
from jax import lax
from jax.experimental import pallas as pl
from jax.experimental.pallas import tpu as pltpu
from jax.experimental.pallas import tpu_sc as plsc

N_NODES = 10000
N_EDGES = 160000
NODE_DIM = 128
EDGE_DIM = 16
HIDDEN_DIM = 128
NUM_RBF = 64
CUTOFF = 10.0

AUXW = 16
NC, NS = 2, 16
NW = NC * NS
CH = 128
NCHUNK = N_EDGES // CH
CPW = (NCHUNK + NW - 1) // NW
NPAD = 10240
RPS = NPAD // NS

EB = 1600
NB = 1000


def _sc_mesh():
    return plsc.VectorSubcoreMesh(core_axis_name="c", subcore_axis_name="s",
                                  num_cores=NC, num_subcores=NS)


def _gather_body(tab, rowi, coli, gi, gj, idxa, idxb, bufa, bufb, sema, semb):
    wid = lax.axis_index("s") * NC + lax.axis_index("c")

    def body(k, carry):
        c = k * NW + wid

        @pl.when(c < NCHUNK)
        def _():
            off = c * CH
            pltpu.sync_copy(rowi.at[pl.ds(off, CH)], idxa)
            pltpu.sync_copy(coli.at[pl.ds(off, CH)], idxb)
            ca = pltpu.async_copy(tab.at[idxa], bufa, sema)
            cb = pltpu.async_copy(tab.at[idxb], bufb, semb)
            ca.wait()
            pltpu.sync_copy(bufa, gi.at[pl.ds(off, CH)])
            cb.wait()
            pltpu.sync_copy(bufb, gj.at[pl.ds(off, CH)])

        return carry

    lax.fori_loop(0, CPW, body, 0)


@functools.cache
def _make_gather(width, tc_tiling):
    return functools.partial(
        pl.kernel,
        out_type=(jax.ShapeDtypeStruct((N_EDGES, width), jnp.float32),
                  jax.ShapeDtypeStruct((N_EDGES, width), jnp.float32)),
        mesh=_sc_mesh(),
        scratch_types=[pltpu.VMEM((CH,), jnp.int32),
                       pltpu.VMEM((CH,), jnp.int32),
                       pltpu.VMEM((CH, width), jnp.float32),
                       pltpu.VMEM((CH, width), jnp.float32),
                       pltpu.SemaphoreType.DMA,
                       pltpu.SemaphoreType.DMA],
        compiler_params=pltpu.CompilerParams(use_tc_tiling_on_sc=tc_tiling),
    )(_gather_body)


def _scatter_body(rowi, val, zz, po, idx, vbuf, acc):
    cid = lax.axis_index("c")
    sid = lax.axis_index("s")
    wid = sid * NC + cid
    r0 = sid * RPS
    pltpu.sync_copy(zz.at[pl.ds(r0, RPS)], acc.at[pl.ds(r0, RPS)])
    plsc.subcore_barrier()

    def body(k, carry):
        c = k * NW + wid

        @pl.when(c < NCHUNK)
        def _():
            off = c * CH
            pltpu.sync_copy(rowi.at[pl.ds(off, CH)], idx)
            pltpu.sync_copy(val.at[pl.ds(off, CH)], vbuf)
            pltpu.sync_copy(vbuf, acc.at[idx], add=True)

        return carry

    lax.fori_loop(0, CPW, body, 0)
    plsc.subcore_barrier()
    pltpu.sync_copy(acc.at[pl.ds(r0, RPS)], po.at[cid, pl.ds(r0, RPS)])


@functools.cache
def _make_scatter(width, tc_tiling):
    return functools.partial(
        pl.kernel,
        out_type=jax.ShapeDtypeStruct((NC, NPAD, width), jnp.float32),
        mesh=_sc_mesh(),
        scratch_types=[pltpu.VMEM((CH,), jnp.int32),
                       pltpu.VMEM((CH, width), jnp.float32),
                       pltpu.VMEM_SHARED((NPAD, width), jnp.float32)],
        compiler_params=pltpu.CompilerParams(use_tc_tiling_on_sc=tc_tiling),
    )(_scatter_body)


def _silu(v):
    return v * jax.nn.sigmoid(v)


def _ln(v, g, b):
    mu = jnp.mean(v, axis=-1, keepdims=True)
    d = v - mu
    var = jnp.mean(d * d, axis=-1, keepdims=True)
    return d * jax.lax.rsqrt(var + 1e-5) * g + b


def _dot(a, b):
    return jax.lax.dot_general(a, b, (((1,), (0,)), ((), ())),
                               preferred_element_type=jnp.float32)


def _edge_body(xi_r, xj_r, ai_r, aj_r, ea, uu,
               w1x, w1y, w1r, w1a, w1u, b1, lg1, lb1, w2, b2, w3, b3,
               mw1x, mw1y, mw1e, mw1r, mb1, mlg, mlb, mw2, mb2, aw, ab,
               cw1x, cw1y, cw1r, cb1, cw2, cb2,
               c5, means, betas,
               ean_o, me_o, aux_o):
    f32 = jnp.float32
    xi = xi_r[...]
    xj = xj_r[...]
    rel = ai_r[:, 0:3] - aj_r[:, 0:3]
    d2 = jnp.sum(rel * rel, axis=-1, keepdims=True) + 1e-12
    dist = jnp.sqrt(d2)
    expd = jnp.exp((-5.0 / CUTOFF) * dist)
    cut = 0.5 * (jnp.cos(dist * (math.pi / CUTOFF)) + 1.0)
    cut = cut * (dist < CUTOFF).astype(f32)
    rbf = cut * jnp.exp(-betas[...] * (expd - means[...]) ** 2)

    h = (_dot(xi, w1x[...]) + _dot(xj, w1y[...]) + _dot(rbf, w1r[...])
         + _dot(ea[...], w1a[...]) + _dot(uu[...], w1u[...]) + b1[...])
    h = _ln(_silu(h), lg1[...], lb1[...])
    h = _silu(_dot(h, w2[...]) + b2[...])
    ean = _dot(h, w3[...]) + b3[...]
    ean_o[...] = ean

    m = (_dot(xi, mw1x[...]) + _dot(xj, mw1y[...]) + _dot(ean, mw1e[...])
         + _dot(rbf, mw1r[...]) + mb1[...])
    m = _ln(_silu(m), mlg[...], mlb[...])
    m = _dot(m, mw2[...]) + mb2[...]

    logits = jnp.sum(m * aw[...], axis=-1, keepdims=True) + ab[...]
    ti = ai_r[:, 3:4]
    tj = aj_r[:, 3:4]
    io5 = lax.broadcasted_iota(jnp.int32, (1, 5), 1)
    ohi = (ti.astype(jnp.int32) == io5).astype(f32)
    ohj = (tj.astype(jnp.int32) == io5).astype(f32)
    bias = jnp.sum(_dot(ohi, c5[...]) * ohj, axis=-1, keepdims=True)
    e = jnp.exp(logits + bias)
    me_o[...] = m * e

    chh = _silu(_dot(xi, cw1x[...]) + _dot(xj, cw1y[...])
                + _dot(rbf, cw1r[...]) + cb1[...])
    w = jnp.tanh(jnp.sum(chh * cw2[...], axis=-1, keepdims=True) + cb2[...])
    one = jnp.ones_like(e)
    aux_o[...] = jnp.concatenate(
        [e, rel * w, one, jnp.zeros((e.shape[0], AUXW - 5), f32)], axis=1)


def _node_body(x, pos, pm, pa,
               nw1x, nw1a, nb1, nlg, nlb, nw2, nb2, ng, nbb,
               xo, po):
    msum = pm[0] + pm[1]
    aux = pa[0] + pa[1]
    denom = aux[:, 0:1]
    delta = aux[:, 1:4]
    cnt = aux[:, 4:5]
    safe = jnp.where(denom > 0.0, denom, 1.0)
    agg = jnp.where(denom > 0.0, msum / safe, 0.0)
    xv = x[...]
    nh = _silu(_dot(xv, nw1x[...]) + _dot(agg, nw1a[...]) + nb1[...])
    nh = _ln(nh, nlg[...], nlb[...])
    nh = _dot(nh, nw2[...]) + nb2[...]
    xo[...] = _ln(xv + nh, ng[...], nbb[...])
    po[...] = pos[...] + delta / (cnt + 1e-8)


def _row(v):
    return v[None, :]


def kernel(x, pos, edge_attr, u, params, edge_index, motif_types):
    p = params
    f32 = jnp.float32
    row = edge_index[0]
    col = edge_index[1]

    atab = jnp.concatenate(
        [pos, motif_types.astype(f32)[:, None],
         jnp.zeros((N_NODES, AUXW - 4), f32)], axis=1)

    xi, xj = _make_gather(NODE_DIM, True)(x, row, col)
    ai, aj = _make_gather(AUXW, False)(atab, row, col)

    w1 = p['eu_w1']
    mw1 = p['msg_w1']
    cw1 = p['coord_w1']
    c5 = (p['cross_bias'] + p['motif_imp'][:, None] + p['motif_imp'][None, :])

    eargs = (
        w1[0:128], w1[128:256], w1[256:320], w1[320:336], w1[336:464],
        _row(p['eu_b1']), _row(p['eu_ln_g']), _row(p['eu_ln_b']),
        p['eu_w2'], _row(p['eu_b2']), p['eu_w3'], _row(p['eu_b3']),
        mw1[0:128], mw1[128:256], mw1[256:272], mw1[272:336],
        _row(p['msg_b1']), _row(p['msg_ln_g']), _row(p['msg_ln_b']),
        p['msg_w2'], _row(p['msg_b2']),
        p['attn_w'].T, _row(p['attn_b']),
        cw1[0:128], cw1[128:256], cw1[256:320], _row(p['coord_b1']),
        p['coord_w2'].T, _row(p['coord_b2']),
        c5, _row(p['eu_means']), _row(p['eu_betas']),
    )

    nblk = N_EDGES // EB
    dspec = lambda d: pl.BlockSpec((EB, d), lambda i: (i, 0))
    wspec = lambda a: pl.BlockSpec(a.shape, lambda i: (0,) * a.ndim)

    ean, me, aux = pl.pallas_call(
        _edge_body,
        grid=(nblk,),
        in_specs=[dspec(NODE_DIM), dspec(NODE_DIM), dspec(AUXW), dspec(AUXW),
                  dspec(EDGE_DIM), dspec(HIDDEN_DIM)]
                 + [wspec(a) for a in eargs],
        out_specs=[dspec(EDGE_DIM), dspec(HIDDEN_DIM), dspec(AUXW)],
        out_shape=[jax.ShapeDtypeStruct((N_EDGES, EDGE_DIM), f32),
                   jax.ShapeDtypeStruct((N_EDGES, HIDDEN_DIM), f32),
                   jax.ShapeDtypeStruct((N_EDGES, AUXW), f32)],
    )(xi, xj, ai, aj, edge_attr, u, *eargs)

    zm = jnp.zeros((NPAD, HIDDEN_DIM), f32)
    za = jnp.zeros((NPAD, AUXW), f32)
    pm = _make_scatter(HIDDEN_DIM, True)(row, me, zm)
    pa = _make_scatter(AUXW, False)(row, aux, za)

    nargs = (
        p['node_w1'][0:128], p['node_w1'][128:256], _row(p['node_b1']),
        _row(p['node_ln_g']), _row(p['node_ln_b']),
        p['node_w2'], _row(p['node_b2']),
        _row(p['norm_g']), _row(p['norm_b']),
    )
    ndspec = lambda d: pl.BlockSpec((NB, d), lambda i: (i, 0))
    pspec = lambda d: pl.BlockSpec((NC, NB, d), lambda i: (0, i, 0))

    x_out, pos_out = pl.pallas_call(
        _node_body,
        grid=(N_NODES // NB,),
        in_specs=[ndspec(NODE_DIM), ndspec(3), pspec(HIDDEN_DIM), pspec(AUXW)]
                 + [wspec(a) for a in nargs],
        out_specs=[ndspec(NODE_DIM), ndspec(3)],
        out_shape=[jax.ShapeDtypeStruct((N_NODES, NODE_DIM), f32),
                   jax.ShapeDtypeStruct((N_NODES, 3), f32)],
    )(x, pos, pm, pa, *nargs)

    return x_out, pos_out, ean


# speedup vs baseline: 5.3072x; 1.0230x over previous
import functools
import math

import jax
import jax.numpy as jnp
from jax import lax
from jax.experimental import pallas as pl
from jax.experimental.pallas import tpu as pltpu
from jax.experimental.pallas import tpu_sc as plsc

N_NODES = 10000
N_EDGES = 160000
NODE_DIM = 128
EDGE_DIM = 16
HIDDEN_DIM = 128
NUM_RBF = 64
CUTOFF = 10.0

AUXW = 16
NC, NS = 2, 16
NW = NC * NS
CH = 128
NCHUNK = N_EDGES // CH
CPW = (NCHUNK + NW - 1) // NW
NPAD = 10240
RPS = NPAD // NS

EB = 1600
NB = 1000


def _sc_mesh():
    return plsc.VectorSubcoreMesh(core_axis_name="c", subcore_axis_name="s",
                                  num_cores=NC, num_subcores=NS)


def _gather_body(tab, rowi, coli, gi, gj, idxa, idxb, bufa, bufb, sema, semb):
    wid = lax.axis_index("s") * NC + lax.axis_index("c")

    def body(k, carry):
        c = k * NW + wid

        @pl.when(c < NCHUNK)
        def _():
            off = c * CH
            pltpu.sync_copy(rowi.at[pl.ds(off, CH)], idxa)
            pltpu.sync_copy(coli.at[pl.ds(off, CH)], idxb)
            ca = pltpu.async_copy(tab.at[idxa], bufa, sema)
            cb = pltpu.async_copy(tab.at[idxb], bufb, semb)
            ca.wait()
            pltpu.sync_copy(bufa, gi.at[pl.ds(off, CH)])
            cb.wait()
            pltpu.sync_copy(bufb, gj.at[pl.ds(off, CH)])

        return carry

    lax.fori_loop(0, CPW, body, 0)


@functools.cache
def _make_gather(width, tc_tiling):
    return functools.partial(
        pl.kernel,
        out_type=(jax.ShapeDtypeStruct((N_EDGES, width), jnp.float32),
                  jax.ShapeDtypeStruct((N_EDGES, width), jnp.float32)),
        mesh=_sc_mesh(),
        scratch_types=[pltpu.VMEM((CH,), jnp.int32),
                       pltpu.VMEM((CH,), jnp.int32),
                       pltpu.VMEM((CH, width), jnp.float32),
                       pltpu.VMEM((CH, width), jnp.float32),
                       pltpu.SemaphoreType.DMA,
                       pltpu.SemaphoreType.DMA],
        compiler_params=pltpu.CompilerParams(use_tc_tiling_on_sc=tc_tiling),
    )(_gather_body)


def _scatter_body(rowi, val, zz, po, idx, vbuf, acc):
    cid = lax.axis_index("c")
    sid = lax.axis_index("s")
    wid = sid * NC + cid
    r0 = sid * RPS
    pltpu.sync_copy(zz.at[pl.ds(r0, RPS)], acc.at[pl.ds(r0, RPS)])
    plsc.subcore_barrier()

    def body(k, carry):
        c = k * NW + wid

        @pl.when(c < NCHUNK)
        def _():
            off = c * CH
            pltpu.sync_copy(rowi.at[pl.ds(off, CH)], idx)
            pltpu.sync_copy(val.at[pl.ds(off, CH)], vbuf)
            pltpu.sync_copy(vbuf, acc.at[idx], add=True)

        return carry

    lax.fori_loop(0, CPW, body, 0)
    plsc.subcore_barrier()
    pltpu.sync_copy(acc.at[pl.ds(r0, RPS)], po.at[cid, pl.ds(r0, RPS)])


@functools.cache
def _make_scatter(width, tc_tiling):
    return functools.partial(
        pl.kernel,
        out_type=jax.ShapeDtypeStruct((NC, NPAD, width), jnp.float32),
        mesh=_sc_mesh(),
        scratch_types=[pltpu.VMEM((CH,), jnp.int32),
                       pltpu.VMEM((CH, width), jnp.float32),
                       pltpu.VMEM_SHARED((NPAD, width), jnp.float32)],
        compiler_params=pltpu.CompilerParams(use_tc_tiling_on_sc=tc_tiling),
    )(_scatter_body)


def _silu(v):
    return v * jax.nn.sigmoid(v)


def _ln(v, g, b):
    mu = jnp.mean(v, axis=-1, keepdims=True)
    d = v - mu
    var = jnp.mean(d * d, axis=-1, keepdims=True)
    return d * jax.lax.rsqrt(var + 1e-5) * g + b


def _dot(a, b):
    return jax.lax.dot_general(a, b, (((1,), (0,)), ((), ())),
                               preferred_element_type=jnp.float32)


def _db(a, b):
    return _dot(a.astype(jnp.bfloat16), b)


def _edge_body(xi_r, xj_r, ai_r, aj_r, ea, uu,
               w1x, w1y, w1r, w1a, w1u, b1, lg1, lb1, w2, b2, w3, b3,
               mw1x, mw1y, mw1e, mw1r, mb1, mlg, mlb, mw2, mb2, aw, ab,
               cw1x, cw1y, cw1r, cb1, cw2, cb2,
               c5, means, betas,
               ean_o, me_o, aux_o):
    f32 = jnp.float32
    xi = xi_r[...].astype(jnp.bfloat16)
    xj = xj_r[...].astype(jnp.bfloat16)
    rel = ai_r[:, 0:3] - aj_r[:, 0:3]
    d2 = jnp.sum(rel * rel, axis=-1, keepdims=True) + 1e-12
    dist = jnp.sqrt(d2)
    expd = jnp.exp((-5.0 / CUTOFF) * dist)
    cut = 0.5 * (jnp.cos(dist * (math.pi / CUTOFF)) + 1.0)
    cut = cut * (dist < CUTOFF).astype(f32)
    rbf = cut * jnp.exp(-betas[...] * (expd - means[...]) ** 2)

    h = (_dot(xi, w1x[...]) + _dot(xj, w1y[...]) + _db(rbf, w1r[...])
         + _db(ea[...], w1a[...]) + _db(uu[...], w1u[...]) + b1[...])
    h = _ln(_silu(h), lg1[...], lb1[...])
    h = _silu(_db(h, w2[...]) + b2[...])
    ean = _db(h, w3[...]) + b3[...]
    ean_o[...] = ean

    m = (_dot(xi, mw1x[...]) + _dot(xj, mw1y[...]) + _db(ean, mw1e[...])
         + _db(rbf, mw1r[...]) + mb1[...])
    m = _ln(_silu(m), mlg[...], mlb[...])
    m = _db(m, mw2[...]) + mb2[...]

    logits = jnp.sum(m * aw[...], axis=-1, keepdims=True) + ab[...]
    ti = ai_r[:, 3:4]
    tj = aj_r[:, 3:4]
    io5 = lax.broadcasted_iota(jnp.int32, (1, 5), 1)
    ohi = (ti.astype(jnp.int32) == io5).astype(f32)
    ohj = (tj.astype(jnp.int32) == io5).astype(f32)
    bias = jnp.sum(_dot(ohi, c5[...]) * ohj, axis=-1, keepdims=True)
    e = jnp.exp(logits + bias)
    me_o[...] = m * e

    chh = _silu(_dot(xi, cw1x[...]) + _dot(xj, cw1y[...])
                + _db(rbf, cw1r[...]) + cb1[...])
    w = jnp.tanh(jnp.sum(chh * cw2[...], axis=-1, keepdims=True) + cb2[...])
    one = jnp.ones_like(e)
    aux_o[...] = jnp.concatenate(
        [e, rel * w, one, jnp.zeros((e.shape[0], AUXW - 5), f32)], axis=1)


def _node_body(x, pos, pm, pa,
               nw1x, nw1a, nb1, nlg, nlb, nw2, nb2, ng, nbb,
               xo, po):
    msum = pm[0] + pm[1]
    aux = pa[0] + pa[1]
    denom = aux[:, 0:1]
    delta = aux[:, 1:4]
    cnt = aux[:, 4:5]
    safe = jnp.where(denom > 0.0, denom, 1.0)
    agg = jnp.where(denom > 0.0, msum / safe, 0.0)
    xv = x[...]
    nh = _silu(_dot(xv, nw1x[...]) + _dot(agg, nw1a[...]) + nb1[...])
    nh = _ln(nh, nlg[...], nlb[...])
    nh = _dot(nh, nw2[...]) + nb2[...]
    xo[...] = _ln(xv + nh, ng[...], nbb[...])
    po[...] = pos[...] + delta / (cnt + 1e-8)


def _row(v):
    return v[None, :]


def kernel(x, pos, edge_attr, u, params, edge_index, motif_types):
    p = params
    f32 = jnp.float32
    row = edge_index[0]
    col = edge_index[1]

    atab = jnp.concatenate(
        [pos, motif_types.astype(f32)[:, None],
         jnp.zeros((N_NODES, AUXW - 4), f32)], axis=1)

    xi, xj = _make_gather(NODE_DIM, True)(x, row, col)
    ai, aj = _make_gather(AUXW, False)(atab, row, col)

    w1 = p['eu_w1']
    mw1 = p['msg_w1']
    cw1 = p['coord_w1']
    c5 = (p['cross_bias'] + p['motif_imp'][:, None] + p['motif_imp'][None, :])

    b16 = jnp.bfloat16
    eargs = (
        w1[0:128].astype(b16), w1[128:256].astype(b16),
        w1[256:320].astype(b16), w1[320:336].astype(b16),
        w1[336:464].astype(b16),
        _row(p['eu_b1']), _row(p['eu_ln_g']), _row(p['eu_ln_b']),
        p['eu_w2'].astype(b16), _row(p['eu_b2']),
        p['eu_w3'].astype(b16), _row(p['eu_b3']),
        mw1[0:128].astype(b16), mw1[128:256].astype(b16),
        mw1[256:272].astype(b16), mw1[272:336].astype(b16),
        _row(p['msg_b1']), _row(p['msg_ln_g']), _row(p['msg_ln_b']),
        p['msg_w2'].astype(b16), _row(p['msg_b2']),
        p['attn_w'].T, _row(p['attn_b']),
        cw1[0:128].astype(b16), cw1[128:256].astype(b16),
        cw1[256:320].astype(b16), _row(p['coord_b1']),
        p['coord_w2'].T, _row(p['coord_b2']),
        c5, _row(p['eu_means']), _row(p['eu_betas']),
    )

    nblk = N_EDGES // EB
    dspec = lambda d: pl.BlockSpec((EB, d), lambda i: (i, 0))
    wspec = lambda a: pl.BlockSpec(a.shape, lambda i: (0,) * a.ndim)

    ean, me, aux = pl.pallas_call(
        _edge_body,
        grid=(nblk,),
        in_specs=[dspec(NODE_DIM), dspec(NODE_DIM), dspec(AUXW), dspec(AUXW),
                  dspec(EDGE_DIM), dspec(HIDDEN_DIM)]
                 + [wspec(a) for a in eargs],
        out_specs=[dspec(EDGE_DIM), dspec(HIDDEN_DIM), dspec(AUXW)],
        out_shape=[jax.ShapeDtypeStruct((N_EDGES, EDGE_DIM), f32),
                   jax.ShapeDtypeStruct((N_EDGES, HIDDEN_DIM), f32),
                   jax.ShapeDtypeStruct((N_EDGES, AUXW), f32)],
    )(xi, xj, ai, aj, edge_attr, u, *eargs)

    zm = jnp.zeros((NPAD, HIDDEN_DIM), f32)
    za = jnp.zeros((NPAD, AUXW), f32)
    pm = _make_scatter(HIDDEN_DIM, True)(row, me, zm)
    pa = _make_scatter(AUXW, False)(row, aux, za)

    nargs = (
        p['node_w1'][0:128], p['node_w1'][128:256], _row(p['node_b1']),
        _row(p['node_ln_g']), _row(p['node_ln_b']),
        p['node_w2'], _row(p['node_b2']),
        _row(p['norm_g']), _row(p['norm_b']),
    )
    ndspec = lambda d: pl.BlockSpec((NB, d), lambda i: (i, 0))
    pspec = lambda d: pl.BlockSpec((NC, NB, d), lambda i: (0, i, 0))

    x_out, pos_out = pl.pallas_call(
        _node_body,
        grid=(N_NODES // NB,),
        in_specs=[ndspec(NODE_DIM), ndspec(3), pspec(HIDDEN_DIM), pspec(AUXW)]
                 + [wspec(a) for a in nargs],
        out_specs=[ndspec(NODE_DIM), ndspec(3)],
        out_shape=[jax.ShapeDtypeStruct((N_NODES, NODE_DIM), f32),
                   jax.ShapeDtypeStruct((N_NODES, 3), f32)],
    )(x, pos, pm, pa, *nargs)

    return x_out, pos_out, ean


# speedup vs baseline: 5.5575x; 1.0472x over previous
import functools
import math

import jax
import jax.numpy as jnp
from jax import lax
from jax.experimental import pallas as pl
from jax.experimental.pallas import tpu as pltpu
from jax.experimental.pallas import tpu_sc as plsc

N_NODES = 10000
N_EDGES = 160000
NODE_DIM = 128
EDGE_DIM = 16
HIDDEN_DIM = 128
NUM_RBF = 64
CUTOFF = 10.0

AUXW = 16
NC, NS = 2, 16
NW = NC * NS
CH = 128
NCHUNK = N_EDGES // CH
CPW = (NCHUNK + NW - 1) // NW
NPAD = 10240
RPS = NPAD // NS

EB = 1600
NB = 1000
NSEG = 2
SEG = N_EDGES // NSEG


def _sc_mesh():
    return plsc.VectorSubcoreMesh(core_axis_name="c", subcore_axis_name="s",
                                  num_cores=NC, num_subcores=NS)


@functools.cache
def _make_gather(width, tc_tiling, ne):
    nchunk = ne // CH
    cpw = (nchunk + NW - 1) // NW

    def gather_body(tab, rowi, coli, gi, gj, idxa, idxb, bufa, bufb, sema,
                    semb):
        wid = lax.axis_index("s") * NC + lax.axis_index("c")

        def body(k, carry):
            c = k * NW + wid

            @pl.when(c < nchunk)
            def _():
                off = c * CH
                pltpu.sync_copy(rowi.at[pl.ds(off, CH)], idxa)
                pltpu.sync_copy(coli.at[pl.ds(off, CH)], idxb)
                ca = pltpu.async_copy(tab.at[idxa], bufa, sema)
                cb = pltpu.async_copy(tab.at[idxb], bufb, semb)
                ca.wait()
                pltpu.sync_copy(bufa, gi.at[pl.ds(off, CH)])
                cb.wait()
                pltpu.sync_copy(bufb, gj.at[pl.ds(off, CH)])

            return carry

        lax.fori_loop(0, cpw, body, 0)

    return functools.partial(
        pl.kernel,
        out_type=(jax.ShapeDtypeStruct((ne, width), jnp.float32),
                  jax.ShapeDtypeStruct((ne, width), jnp.float32)),
        mesh=_sc_mesh(),
        scratch_types=[pltpu.VMEM((CH,), jnp.int32),
                       pltpu.VMEM((CH,), jnp.int32),
                       pltpu.VMEM((CH, width), jnp.float32),
                       pltpu.VMEM((CH, width), jnp.float32),
                       pltpu.SemaphoreType.DMA,
                       pltpu.SemaphoreType.DMA],
        compiler_params=pltpu.CompilerParams(use_tc_tiling_on_sc=tc_tiling),
    )(gather_body)


@functools.cache
def _make_scatter(width, tc_tiling, ne):
    nchunk = ne // CH
    cpw = (nchunk + NW - 1) // NW

    def scatter_body(rowi, val, zz, po, idx, vbuf, acc):
        cid = lax.axis_index("c")
        sid = lax.axis_index("s")
        wid = sid * NC + cid
        r0 = sid * RPS
        pltpu.sync_copy(zz.at[pl.ds(r0, RPS)], acc.at[pl.ds(r0, RPS)])
        plsc.subcore_barrier()

        def body(k, carry):
            c = k * NW + wid

            @pl.when(c < nchunk)
            def _():
                off = c * CH
                pltpu.sync_copy(rowi.at[pl.ds(off, CH)], idx)
                pltpu.sync_copy(val.at[pl.ds(off, CH)], vbuf)
                pltpu.sync_copy(vbuf, acc.at[idx], add=True)

            return carry

        lax.fori_loop(0, cpw, body, 0)
        plsc.subcore_barrier()
        pltpu.sync_copy(acc.at[pl.ds(r0, RPS)], po.at[cid, pl.ds(r0, RPS)])

    return functools.partial(
        pl.kernel,
        out_type=jax.ShapeDtypeStruct((NC, NPAD, width), jnp.float32),
        mesh=_sc_mesh(),
        scratch_types=[pltpu.VMEM((CH,), jnp.int32),
                       pltpu.VMEM((CH, width), jnp.float32),
                       pltpu.VMEM_SHARED((NPAD, width), jnp.float32)],
        compiler_params=pltpu.CompilerParams(use_tc_tiling_on_sc=tc_tiling),
    )(scatter_body)


def _silu(v):
    return v * jax.nn.sigmoid(v)


def _ln(v, g, b):
    mu = jnp.mean(v, axis=-1, keepdims=True)
    d = v - mu
    var = jnp.mean(d * d, axis=-1, keepdims=True)
    return d * jax.lax.rsqrt(var + 1e-5) * g + b


def _dot(a, b):
    return jax.lax.dot_general(a, b, (((1,), (0,)), ((), ())),
                               preferred_element_type=jnp.float32)


def _db(a, b):
    return _dot(a.astype(jnp.bfloat16), b)


def _edge_body(xi_r, xj_r, ai_r, aj_r, ea, uu,
               w1x, w1y, w1r, w1a, w1u, b1, lg1, lb1, w2, b2, w3, b3,
               mw1x, mw1y, mw1e, mw1r, mb1, mlg, mlb, mw2, mb2, aw, ab,
               cw1x, cw1y, cw1r, cb1, cw2, cb2,
               c5, means, betas,
               ean_o, me_o, aux_o):
    f32 = jnp.float32
    xi = xi_r[...].astype(jnp.bfloat16)
    xj = xj_r[...].astype(jnp.bfloat16)
    rel = ai_r[:, 0:3] - aj_r[:, 0:3]
    d2 = jnp.sum(rel * rel, axis=-1, keepdims=True) + 1e-12
    dist = jnp.sqrt(d2)
    expd = jnp.exp((-5.0 / CUTOFF) * dist)
    cut = 0.5 * (jnp.cos(dist * (math.pi / CUTOFF)) + 1.0)
    cut = cut * (dist < CUTOFF).astype(f32)
    rbf = cut * jnp.exp(-betas[...] * (expd - means[...]) ** 2)

    h = (_dot(xi, w1x[...]) + _dot(xj, w1y[...]) + _db(rbf, w1r[...])
         + _db(ea[...], w1a[...]) + _db(uu[...], w1u[...]) + b1[...])
    h = _ln(_silu(h), lg1[...], lb1[...])
    h = _silu(_db(h, w2[...]) + b2[...])
    ean = _db(h, w3[...]) + b3[...]
    ean_o[...] = ean

    m = (_dot(xi, mw1x[...]) + _dot(xj, mw1y[...]) + _db(ean, mw1e[...])
         + _db(rbf, mw1r[...]) + mb1[...])
    m = _ln(_silu(m), mlg[...], mlb[...])
    m = _db(m, mw2[...]) + mb2[...]

    logits = jnp.sum(m * aw[...], axis=-1, keepdims=True) + ab[...]
    ti = ai_r[:, 3:4]
    tj = aj_r[:, 3:4]
    io5 = lax.broadcasted_iota(jnp.int32, (1, 5), 1)
    ohi = (ti.astype(jnp.int32) == io5).astype(f32)
    ohj = (tj.astype(jnp.int32) == io5).astype(f32)
    bias = jnp.sum(_dot(ohi, c5[...]) * ohj, axis=-1, keepdims=True)
    e = jnp.exp(logits + bias)
    me_o[...] = m * e

    chh = _silu(_dot(xi, cw1x[...]) + _dot(xj, cw1y[...])
                + _db(rbf, cw1r[...]) + cb1[...])
    w = jnp.tanh(jnp.sum(chh * cw2[...], axis=-1, keepdims=True) + cb2[...])
    one = jnp.ones_like(e)
    aux_o[...] = jnp.concatenate(
        [e, rel * w, one, jnp.zeros((e.shape[0], AUXW - 5), f32)], axis=1)


def _node_body(x, pos, pm, pa,
               nw1x, nw1a, nb1, nlg, nlb, nw2, nb2, ng, nbb,
               xo, po):
    msum = pm[0]
    aux = pa[0]
    for k in range(1, pm.shape[0]):
        msum = msum + pm[k]
        aux = aux + pa[k]
    denom = aux[:, 0:1]
    delta = aux[:, 1:4]
    cnt = aux[:, 4:5]
    safe = jnp.where(denom > 0.0, denom, 1.0)
    agg = jnp.where(denom > 0.0, msum / safe, 0.0)
    xv = x[...]
    nh = _silu(_dot(xv, nw1x[...]) + _dot(agg, nw1a[...]) + nb1[...])
    nh = _ln(nh, nlg[...], nlb[...])
    nh = _dot(nh, nw2[...]) + nb2[...]
    xo[...] = _ln(xv + nh, ng[...], nbb[...])
    po[...] = pos[...] + delta / (cnt + 1e-8)


def _row(v):
    return v[None, :]


def kernel(x, pos, edge_attr, u, params, edge_index, motif_types):
    p = params
    f32 = jnp.float32
    row = edge_index[0]
    col = edge_index[1]

    atab = jnp.concatenate(
        [pos, motif_types.astype(f32)[:, None],
         jnp.zeros((N_NODES, AUXW - 4), f32)], axis=1)


    w1 = p['eu_w1']
    mw1 = p['msg_w1']
    cw1 = p['coord_w1']
    c5 = (p['cross_bias'] + p['motif_imp'][:, None] + p['motif_imp'][None, :])

    b16 = jnp.bfloat16
    eargs = (
        w1[0:128].astype(b16), w1[128:256].astype(b16),
        w1[256:320].astype(b16), w1[320:336].astype(b16),
        w1[336:464].astype(b16),
        _row(p['eu_b1']), _row(p['eu_ln_g']), _row(p['eu_ln_b']),
        p['eu_w2'].astype(b16), _row(p['eu_b2']),
        p['eu_w3'].astype(b16), _row(p['eu_b3']),
        mw1[0:128].astype(b16), mw1[128:256].astype(b16),
        mw1[256:272].astype(b16), mw1[272:336].astype(b16),
        _row(p['msg_b1']), _row(p['msg_ln_g']), _row(p['msg_ln_b']),
        p['msg_w2'].astype(b16), _row(p['msg_b2']),
        p['attn_w'].T, _row(p['attn_b']),
        cw1[0:128].astype(b16), cw1[128:256].astype(b16),
        cw1[256:320].astype(b16), _row(p['coord_b1']),
        p['coord_w2'].T, _row(p['coord_b2']),
        c5, _row(p['eu_means']), _row(p['eu_betas']),
    )

    wspec = lambda a: pl.BlockSpec(a.shape, lambda i: (0,) * a.ndim)
    gx = _make_gather(NODE_DIM, True, SEG)
    ga = _make_gather(AUXW, False, SEG)
    sm = _make_scatter(HIDDEN_DIM, True, SEG)
    sa = _make_scatter(AUXW, False, SEG)
    zm = jnp.zeros((NPAD, HIDDEN_DIM), f32)
    za = jnp.zeros((NPAD, AUXW), f32)

    rows = [row[i * SEG:(i + 1) * SEG] for i in range(NSEG)]
    cols = [col[i * SEG:(i + 1) * SEG] for i in range(NSEG)]
    gx_out = [gx(x, rows[i], cols[i]) for i in range(NSEG)]
    ga_out = [ga(atab, rows[i], cols[i]) for i in range(NSEG)]

    nblk = SEG // EB
    eans, pms, pas = [], [], []
    for si in range(NSEG):
        base = si * nblk
        dspec = lambda d: pl.BlockSpec((EB, d), lambda i: (i, 0))
        fspec = lambda d: pl.BlockSpec((EB, d), lambda i, b=base: (b + i, 0))
        xi, xj = gx_out[si]
        ai, aj = ga_out[si]
        ean_s, me_s, aux_s = pl.pallas_call(
            _edge_body,
            grid=(nblk,),
            in_specs=[dspec(NODE_DIM), dspec(NODE_DIM), dspec(AUXW),
                      dspec(AUXW), fspec(EDGE_DIM), fspec(HIDDEN_DIM)]
                     + [wspec(a) for a in eargs],
            out_specs=[dspec(EDGE_DIM), dspec(HIDDEN_DIM), dspec(AUXW)],
            out_shape=[jax.ShapeDtypeStruct((SEG, EDGE_DIM), f32),
                       jax.ShapeDtypeStruct((SEG, HIDDEN_DIM), f32),
                       jax.ShapeDtypeStruct((SEG, AUXW), f32)],
        )(xi, xj, ai, aj, edge_attr, u, *eargs)
        eans.append(ean_s)
        pms.append(sm(rows[si], me_s, zm))
        pas.append(sa(rows[si], aux_s, za))

    ean = jnp.concatenate(eans, axis=0)
    pm = jnp.concatenate(pms, axis=0)
    pa = jnp.concatenate(pas, axis=0)

    nargs = (
        p['node_w1'][0:128], p['node_w1'][128:256], _row(p['node_b1']),
        _row(p['node_ln_g']), _row(p['node_ln_b']),
        p['node_w2'], _row(p['node_b2']),
        _row(p['norm_g']), _row(p['norm_b']),
    )
    ndspec = lambda d: pl.BlockSpec((NB, d), lambda i: (i, 0))
    pspec = lambda d: pl.BlockSpec((NC * NSEG, NB, d), lambda i: (0, i, 0))

    x_out, pos_out = pl.pallas_call(
        _node_body,
        grid=(N_NODES // NB,),
        in_specs=[ndspec(NODE_DIM), ndspec(3), pspec(HIDDEN_DIM), pspec(AUXW)]
                 + [wspec(a) for a in nargs],
        out_specs=[ndspec(NODE_DIM), ndspec(3)],
        out_shape=[jax.ShapeDtypeStruct((N_NODES, NODE_DIM), f32),
                   jax.ShapeDtypeStruct((N_NODES, 3), f32)],
    )(x, pos, pm, pa, *nargs)

    return x_out, pos_out, ean
